# baseline (speedup 1.0000x reference)
import jax
import jax.numpy as jnp
from jax import lax
from jax.experimental import pallas as pl
from jax.experimental.pallas import tpu as pltpu, tpu_sc as plsc

NUM_NODES = 10000
NUM_EDGES = 320000
HIDDEN = 128
RADIAL = 6
LANES = 16
HVECS = HIDDEN // LANES

NC = 2
NS = 16
EDGES_PER_CORE = NUM_EDGES // NC
EDGES_PER_SUB = EDGES_PER_CORE // NS
CHUNK = 200
NUM_CHUNKS = EDGES_PER_SUB // CHUNK
ROWS_MAIN = 640
ROWS_TAIL = NUM_NODES - (NS - 1) * ROWS_MAIN

NODE_BLOCK = 1000


def _fused_body(x_hbm, rbf_hbm, i_hbm, z_hbm, wt_hbm, out_hbm,
                idx_v, xbuf, rbf_v, wt_v, acc_sh):
    c = lax.axis_index("c")
    s = lax.axis_index("s")

    pltpu.sync_copy(wt_hbm, wt_v)

    @pl.when(s < NS - 1)
    def _():
        pltpu.sync_copy(
            z_hbm.at[pl.ds(s * ROWS_MAIN, ROWS_MAIN)],
            acc_sh.at[pl.ds(s * ROWS_MAIN, ROWS_MAIN)],
        )

    @pl.when(s == NS - 1)
    def _():
        pltpu.sync_copy(
            z_hbm.at[pl.ds((NS - 1) * ROWS_MAIN, ROWS_TAIL)],
            acc_sh.at[pl.ds((NS - 1) * ROWS_MAIN, ROWS_TAIL)],
        )

    plsc.subcore_barrier()

    base0 = c * EDGES_PER_CORE + s * EDGES_PER_SUB

    def chunk_body(k, _):
        base = base0 + k * CHUNK
        pltpu.sync_copy(i_hbm.at[pl.ds(base, CHUNK)], idx_v)
        pltpu.sync_copy(
            rbf_hbm.at[pl.ds(base * RADIAL, CHUNK * RADIAL)],
            rbf_v.at[pl.ds(0, CHUNK * RADIAL)],
        )
        pltpu.sync_copy(x_hbm.at[pl.ds(base, CHUNK)], xbuf)

        def edge(e, _):
            rb = e * RADIAL
            rvec = rbf_v[pl.ds(rb, LANES)]
            taps = [rvec[r] for r in range(RADIAL)]
            for j in range(HVECS):
                sv = taps[0] * wt_v[0, pl.ds(j * LANES, LANES)]
                for r in range(1, RADIAL):
                    sv = sv + taps[r] * wt_v[r, pl.ds(j * LANES, LANES)]
                xbuf[e, pl.ds(j * LANES, LANES)] = (
                    sv * xbuf[e, pl.ds(j * LANES, LANES)]
                )
            return _

        lax.fori_loop(0, CHUNK, edge, None)

        pltpu.sync_copy(xbuf, acc_sh.at[idx_v], add=True)
        return _

    lax.fori_loop(0, NUM_CHUNKS, chunk_body, None)

    plsc.subcore_barrier()

    @pl.when(s < NS - 1)
    def _():
        pltpu.sync_copy(
            acc_sh.at[pl.ds(s * ROWS_MAIN, ROWS_MAIN)],
            out_hbm.at[c, pl.ds(s * ROWS_MAIN, ROWS_MAIN)],
        )

    @pl.when(s == NS - 1)
    def _():
        pltpu.sync_copy(
            acc_sh.at[pl.ds((NS - 1) * ROWS_MAIN, ROWS_TAIL)],
            out_hbm.at[c, pl.ds((NS - 1) * ROWS_MAIN, ROWS_TAIL)],
        )


_fused_stage = pl.kernel(
    _fused_body,
    out_type=jax.ShapeDtypeStruct((NC, NUM_NODES, HIDDEN), jnp.float32),
    mesh=plsc.VectorSubcoreMesh(core_axis_name="c", subcore_axis_name="s"),
    scratch_types=[
        pltpu.VMEM((CHUNK,), jnp.int32),
        pltpu.VMEM((CHUNK, HIDDEN), jnp.float32),
        pltpu.VMEM((CHUNK * RADIAL + LANES,), jnp.float32),
        pltpu.VMEM((RADIAL, HIDDEN), jnp.float32),
        pltpu.VMEM_SHARED((NUM_NODES, HIDDEN), jnp.float32),
    ],
)


def _mlp_body(parts_ref, wup_ref, wl_ref, bl_ref, wout_ref, out_ref):
    xt = parts_ref[0] + parts_ref[1]
    h = lax.dot_general(
        xt, wup_ref[...], (((1,), (1,)), ((), ())),
        preferred_element_type=jnp.float32,
    )
    for l in range(wl_ref.shape[0]):
        z = lax.dot_general(
            h, wl_ref[l], (((1,), (1,)), ((), ())),
            preferred_element_type=jnp.float32,
        ) + bl_ref[l][None, :]
        h = z * jax.nn.sigmoid(z)
    out_ref[...] = lax.dot_general(
        h, wout_ref[...], (((1,), (1,)), ((), ())),
        preferred_element_type=jnp.float32,
    )


def _mlp_stage(parts, w_up, w_layers, b_layers, w_out):
    grid = (NUM_NODES // NODE_BLOCK,)
    return pl.pallas_call(
        _mlp_body,
        grid=grid,
        in_specs=[
            pl.BlockSpec((NC, NODE_BLOCK, HIDDEN), lambda j: (0, j, 0)),
            pl.BlockSpec(w_up.shape, lambda j: (0, 0)),
            pl.BlockSpec(w_layers.shape, lambda j: (0, 0, 0)),
            pl.BlockSpec(b_layers.shape, lambda j: (0, 0)),
            pl.BlockSpec(w_out.shape, lambda j: (0, 0)),
        ],
        out_specs=pl.BlockSpec((NODE_BLOCK, w_out.shape[0]), lambda j: (j, 0)),
        out_shape=jax.ShapeDtypeStruct((NUM_NODES, w_out.shape[0]), jnp.float32),
    )(parts, w_up, w_layers, b_layers, w_out)


def kernel(x, rbf, i, num_nodes, W_rbfs, W_up, W_layers, b_layers, W_out):
    wt = jnp.transpose(W_rbfs[-1])
    rbf_flat = jnp.reshape(rbf, (-1,))
    zeros = jnp.zeros((NUM_NODES, HIDDEN), jnp.float32)
    parts = _fused_stage(x, rbf_flat, i, zeros, wt)
    return _mlp_stage(parts, W_up, W_layers, b_layers, W_out)


# speedup vs baseline: 2.0539x; 2.0539x over previous
import jax
import jax.numpy as jnp
from jax import lax
from jax.experimental import pallas as pl
from jax.experimental.pallas import tpu as pltpu, tpu_sc as plsc

NUM_NODES = 10000
NUM_EDGES = 320000
HIDDEN = 128
RADIAL = 6
LANES = 16
HVECS = HIDDEN // LANES

NC = 2
NS = 16
EDGES_PER_CORE = NUM_EDGES // NC
EDGES_PER_SUB = EDGES_PER_CORE // NS
CHUNK = 200
NUM_CHUNKS = EDGES_PER_SUB // CHUNK
ROWS_MAIN = 640
ROWS_TAIL = NUM_NODES - (NS - 1) * ROWS_MAIN

NODE_BLOCK = 1000


def _fused_body(x_hbm, rbf_hbm, i_hbm, z_hbm, wt_hbm, out_hbm,
                idx_v, xbuf, rbf_v, wt_v, acc_sh):
    c = lax.axis_index("c")
    s = lax.axis_index("s")

    pltpu.sync_copy(wt_hbm, wt_v)

    @pl.when(s < NS - 1)
    def _():
        pltpu.sync_copy(
            z_hbm.at[pl.ds(s * ROWS_MAIN, ROWS_MAIN)],
            acc_sh.at[pl.ds(s * ROWS_MAIN, ROWS_MAIN)],
        )

    @pl.when(s == NS - 1)
    def _():
        pltpu.sync_copy(
            z_hbm.at[pl.ds((NS - 1) * ROWS_MAIN, ROWS_TAIL)],
            acc_sh.at[pl.ds((NS - 1) * ROWS_MAIN, ROWS_TAIL)],
        )

    plsc.subcore_barrier()

    base0 = c * EDGES_PER_CORE + s * EDGES_PER_SUB

    wv = [[wt_v[r, pl.ds(j * LANES, LANES)] for r in range(RADIAL)]
          for j in range(HVECS)]

    def chunk_body(k, _):
        base = base0 + k * CHUNK
        pltpu.sync_copy(i_hbm.at[pl.ds(base, CHUNK)], idx_v)
        pltpu.sync_copy(
            rbf_hbm.at[pl.ds(base * RADIAL, CHUNK * RADIAL)],
            rbf_v.at[pl.ds(0, CHUNK * RADIAL)],
        )
        pltpu.sync_copy(x_hbm.at[pl.ds(base, CHUNK)], xbuf)

        def edge(e, _):
            rb = e * RADIAL
            rvec = rbf_v[pl.ds(rb, LANES)]
            taps = [rvec[r] for r in range(RADIAL)]
            for j in range(HVECS):
                p0 = taps[0] * wv[j][0] + taps[1] * wv[j][1]
                p1 = taps[2] * wv[j][2] + taps[3] * wv[j][3]
                p2 = taps[4] * wv[j][4] + taps[5] * wv[j][5]
                sv = (p0 + p1) + p2
                xbuf[e, pl.ds(j * LANES, LANES)] = (
                    sv * xbuf[e, pl.ds(j * LANES, LANES)]
                )
            return _

        lax.fori_loop(0, CHUNK, edge, None)

        pltpu.sync_copy(xbuf, acc_sh.at[idx_v], add=True)
        return _

    lax.fori_loop(0, NUM_CHUNKS, chunk_body, None)

    plsc.subcore_barrier()

    @pl.when(s < NS - 1)
    def _():
        pltpu.sync_copy(
            acc_sh.at[pl.ds(s * ROWS_MAIN, ROWS_MAIN)],
            out_hbm.at[c, pl.ds(s * ROWS_MAIN, ROWS_MAIN)],
        )

    @pl.when(s == NS - 1)
    def _():
        pltpu.sync_copy(
            acc_sh.at[pl.ds((NS - 1) * ROWS_MAIN, ROWS_TAIL)],
            out_hbm.at[c, pl.ds((NS - 1) * ROWS_MAIN, ROWS_TAIL)],
        )


_fused_stage = pl.kernel(
    _fused_body,
    out_type=jax.ShapeDtypeStruct((NC, NUM_NODES, HIDDEN), jnp.float32),
    mesh=plsc.VectorSubcoreMesh(core_axis_name="c", subcore_axis_name="s"),
    scratch_types=[
        pltpu.VMEM((CHUNK,), jnp.int32),
        pltpu.VMEM((CHUNK, HIDDEN), jnp.float32),
        pltpu.VMEM((CHUNK * RADIAL + LANES,), jnp.float32),
        pltpu.VMEM((RADIAL, HIDDEN), jnp.float32),
        pltpu.VMEM_SHARED((NUM_NODES, HIDDEN), jnp.float32),
    ],
)


def _mlp_body(parts_ref, wup_ref, wl_ref, bl_ref, wout_ref, out_ref):
    xt = parts_ref[0] + parts_ref[1]
    h = lax.dot_general(
        xt, wup_ref[...], (((1,), (1,)), ((), ())),
        preferred_element_type=jnp.float32,
    )
    for l in range(wl_ref.shape[0]):
        z = lax.dot_general(
            h, wl_ref[l], (((1,), (1,)), ((), ())),
            preferred_element_type=jnp.float32,
        ) + bl_ref[l][None, :]
        h = z * jax.nn.sigmoid(z)
    out_ref[...] = lax.dot_general(
        h, wout_ref[...], (((1,), (1,)), ((), ())),
        preferred_element_type=jnp.float32,
    )


def _mlp_stage(parts, w_up, w_layers, b_layers, w_out):
    grid = (NUM_NODES // NODE_BLOCK,)
    return pl.pallas_call(
        _mlp_body,
        grid=grid,
        in_specs=[
            pl.BlockSpec((NC, NODE_BLOCK, HIDDEN), lambda j: (0, j, 0)),
            pl.BlockSpec(w_up.shape, lambda j: (0, 0)),
            pl.BlockSpec(w_layers.shape, lambda j: (0, 0, 0)),
            pl.BlockSpec(b_layers.shape, lambda j: (0, 0)),
            pl.BlockSpec(w_out.shape, lambda j: (0, 0)),
        ],
        out_specs=pl.BlockSpec((NODE_BLOCK, w_out.shape[0]), lambda j: (j, 0)),
        out_shape=jax.ShapeDtypeStruct((NUM_NODES, w_out.shape[0]), jnp.float32),
    )(parts, w_up, w_layers, b_layers, w_out)


def kernel(x, rbf, i, num_nodes, W_rbfs, W_up, W_layers, b_layers, W_out):
    wt = jnp.transpose(W_rbfs[-1])
    rbf_flat = jnp.reshape(rbf, (-1,))
    zeros = jnp.zeros((NUM_NODES, HIDDEN), jnp.float32)
    parts = _fused_stage(x, rbf_flat, i, zeros, wt)
    return _mlp_stage(parts, W_up, W_layers, b_layers, W_out)


# speedup vs baseline: 3.5619x; 1.7342x over previous
import jax
import jax.numpy as jnp
from jax import lax
from jax.experimental import pallas as pl
from jax.experimental.pallas import tpu as pltpu, tpu_sc as plsc

NUM_NODES = 10000
NUM_EDGES = 320000
HIDDEN = 128

NC = 2
NS = 16
EDGES_PER_CORE = NUM_EDGES // NC
EDGES_PER_SUB = EDGES_PER_CORE // NS
CHUNK = 192
TAIL = EDGES_PER_SUB - (EDGES_PER_SUB // CHUNK) * CHUNK
NUM_CHUNKS = EDGES_PER_SUB // CHUNK
ROWS_MAIN = 640
ROWS_TAIL = NUM_NODES - (NS - 1) * ROWS_MAIN

EDGE_BLOCK = 4000
NODE_BLOCK = 1000


def _edge_body(rbf_ref, x_ref, wt_ref, t_ref):
    s = jnp.dot(rbf_ref[...], wt_ref[...], preferred_element_type=jnp.float32)
    t_ref[...] = s * x_ref[...]


def _edge_stage(rbf, x, wt):
    grid = (NUM_EDGES // EDGE_BLOCK,)
    return pl.pallas_call(
        _edge_body,
        grid=grid,
        in_specs=[
            pl.BlockSpec((EDGE_BLOCK, rbf.shape[1]), lambda i: (i, 0)),
            pl.BlockSpec((EDGE_BLOCK, HIDDEN), lambda i: (i, 0)),
            pl.BlockSpec(wt.shape, lambda i: (0, 0)),
        ],
        out_specs=pl.BlockSpec((EDGE_BLOCK, HIDDEN), lambda i: (i, 0)),
        out_shape=jax.ShapeDtypeStruct((NUM_EDGES, HIDDEN), jnp.float32),
    )(rbf, x, wt)


def _scatter_body(t_hbm, i_hbm, z_hbm, out_hbm,
                  idx0, rows0, idx1, rows1, idx_t,
                  sem_i0, sem_r0, sem_i1, sem_r1, acc_sh):
    c = lax.axis_index("c")
    s = lax.axis_index("s")

    @pl.when(s < NS - 1)
    def _():
        pltpu.sync_copy(
            z_hbm.at[pl.ds(s * ROWS_MAIN, ROWS_MAIN)],
            acc_sh.at[pl.ds(s * ROWS_MAIN, ROWS_MAIN)],
        )

    @pl.when(s == NS - 1)
    def _():
        pltpu.sync_copy(
            z_hbm.at[pl.ds((NS - 1) * ROWS_MAIN, ROWS_TAIL)],
            acc_sh.at[pl.ds((NS - 1) * ROWS_MAIN, ROWS_TAIL)],
        )

    plsc.subcore_barrier()

    base0 = c * EDGES_PER_CORE + s * EDGES_PER_SUB

    pltpu.sync_copy(i_hbm.at[pl.ds(base0, TAIL)], idx_t)
    pltpu.sync_copy(t_hbm.at[pl.ds(base0, TAIL)], rows0.at[pl.ds(0, TAIL)])
    pltpu.sync_copy(rows0.at[pl.ds(0, TAIL)], acc_sh.at[idx_t], add=True)

    bufs = ((idx0, rows0, sem_i0, sem_r0), (idx1, rows1, sem_i1, sem_r1))

    def start_load(k, idx_v, rows_v, sem_i, sem_r):
        base = base0 + TAIL + k * CHUNK
        pltpu.async_copy(i_hbm.at[pl.ds(base, CHUNK)], idx_v, sem_i)
        pltpu.async_copy(t_hbm.at[pl.ds(base, CHUNK)], rows_v, sem_r)

    def wait_load(k, idx_v, rows_v, sem_i, sem_r):
        base = base0 + TAIL + k * CHUNK
        pltpu.make_async_copy(i_hbm.at[pl.ds(base, CHUNK)], idx_v, sem_i).wait()
        pltpu.make_async_copy(t_hbm.at[pl.ds(base, CHUNK)], rows_v, sem_r).wait()

    start_load(0, *bufs[0])

    def pair(p, _):
        k0 = 2 * p
        for b in range(2):
            k = k0 + b
            idx_v, rows_v, sem_i, sem_r = bufs[b]
            wait_load(k, idx_v, rows_v, sem_i, sem_r)

            @pl.when(k + 1 < NUM_CHUNKS)
            def _():
                start_load(k + 1, *bufs[1 - b])

            pltpu.sync_copy(rows_v, acc_sh.at[idx_v], add=True)
        return _

    lax.fori_loop(0, NUM_CHUNKS // 2, pair, None)

    plsc.subcore_barrier()

    @pl.when(s < NS - 1)
    def _():
        pltpu.sync_copy(
            acc_sh.at[pl.ds(s * ROWS_MAIN, ROWS_MAIN)],
            out_hbm.at[c, pl.ds(s * ROWS_MAIN, ROWS_MAIN)],
        )

    @pl.when(s == NS - 1)
    def _():
        pltpu.sync_copy(
            acc_sh.at[pl.ds((NS - 1) * ROWS_MAIN, ROWS_TAIL)],
            out_hbm.at[c, pl.ds((NS - 1) * ROWS_MAIN, ROWS_TAIL)],
        )


_scatter_stage = pl.kernel(
    _scatter_body,
    out_type=jax.ShapeDtypeStruct((NC, NUM_NODES, HIDDEN), jnp.float32),
    mesh=plsc.VectorSubcoreMesh(core_axis_name="c", subcore_axis_name="s"),
    scratch_types=[
        pltpu.VMEM((CHUNK,), jnp.int32),
        pltpu.VMEM((CHUNK, HIDDEN), jnp.float32),
        pltpu.VMEM((CHUNK,), jnp.int32),
        pltpu.VMEM((CHUNK, HIDDEN), jnp.float32),
        pltpu.VMEM((TAIL,), jnp.int32),
        pltpu.SemaphoreType.DMA,
        pltpu.SemaphoreType.DMA,
        pltpu.SemaphoreType.DMA,
        pltpu.SemaphoreType.DMA,
        pltpu.VMEM_SHARED((NUM_NODES, HIDDEN), jnp.float32),
    ],
)


def _mlp_body(parts_ref, wup_ref, wl_ref, bl_ref, wout_ref, out_ref):
    xt = parts_ref[0] + parts_ref[1]
    h = lax.dot_general(
        xt, wup_ref[...], (((1,), (1,)), ((), ())),
        preferred_element_type=jnp.float32,
    )
    for l in range(wl_ref.shape[0]):
        z = lax.dot_general(
            h, wl_ref[l], (((1,), (1,)), ((), ())),
            preferred_element_type=jnp.float32,
        ) + bl_ref[l][None, :]
        h = z * jax.nn.sigmoid(z)
    out_ref[...] = lax.dot_general(
        h, wout_ref[...], (((1,), (1,)), ((), ())),
        preferred_element_type=jnp.float32,
    )


def _mlp_stage(parts, w_up, w_layers, b_layers, w_out):
    grid = (NUM_NODES // NODE_BLOCK,)
    return pl.pallas_call(
        _mlp_body,
        grid=grid,
        in_specs=[
            pl.BlockSpec((NC, NODE_BLOCK, HIDDEN), lambda j: (0, j, 0)),
            pl.BlockSpec(w_up.shape, lambda j: (0, 0)),
            pl.BlockSpec(w_layers.shape, lambda j: (0, 0, 0)),
            pl.BlockSpec(b_layers.shape, lambda j: (0, 0)),
            pl.BlockSpec(w_out.shape, lambda j: (0, 0)),
        ],
        out_specs=pl.BlockSpec((NODE_BLOCK, w_out.shape[0]), lambda j: (j, 0)),
        out_shape=jax.ShapeDtypeStruct((NUM_NODES, w_out.shape[0]), jnp.float32),
    )(parts, w_up, w_layers, b_layers, w_out)


def kernel(x, rbf, i, num_nodes, W_rbfs, W_up, W_layers, b_layers, W_out):
    wt = jnp.transpose(W_rbfs[-1])
    t = _edge_stage(rbf, x, wt)
    zeros = jnp.zeros((NUM_NODES, HIDDEN), jnp.float32)
    parts = _scatter_stage(t, i, zeros)
    return _mlp_stage(parts, W_up, W_layers, b_layers, W_out)


# speedup vs baseline: 5.2781x; 1.4818x over previous
import jax
import jax.numpy as jnp
from jax import lax
from jax.experimental import pallas as pl
from jax.experimental.pallas import tpu as pltpu, tpu_sc as plsc

NUM_NODES = 10000
NUM_EDGES = 320000
HIDDEN = 128

NC = 2
NS = 16
EDGES_PER_CORE = NUM_EDGES // NC
EDGES_PER_SUB = EDGES_PER_CORE // NS
CHUNK = 192
TAIL = EDGES_PER_SUB - (EDGES_PER_SUB // CHUNK) * CHUNK
NUM_CHUNKS = EDGES_PER_SUB // CHUNK
ROWS_MAIN = 640
ROWS_TAIL = NUM_NODES - (NS - 1) * ROWS_MAIN

EDGE_BLOCK = 6400
NODE_BLOCK = 1000


def _edge_body(rbft_ref, x_ref, wt_ref, t_ref):
    s = lax.dot_general(
        rbft_ref[...], wt_ref[...], (((0,), (0,)), ((), ())),
        preferred_element_type=jnp.float32,
    )
    t_ref[...] = s * x_ref[...]


def _edge_stage(rbft, x, wt):
    grid = (NUM_EDGES // EDGE_BLOCK,)
    return pl.pallas_call(
        _edge_body,
        grid=grid,
        in_specs=[
            pl.BlockSpec((rbft.shape[0], EDGE_BLOCK), lambda i: (0, i)),
            pl.BlockSpec((EDGE_BLOCK, HIDDEN), lambda i: (i, 0)),
            pl.BlockSpec(wt.shape, lambda i: (0, 0)),
        ],
        out_specs=pl.BlockSpec((EDGE_BLOCK, HIDDEN), lambda i: (i, 0)),
        out_shape=jax.ShapeDtypeStruct((NUM_EDGES, HIDDEN), jnp.float32),
    )(rbft, x, wt)


def _scatter_body(t_hbm, i_hbm, z_hbm, out_hbm,
                  idx0, rows0, idx1, rows1, idx_t,
                  sem_i0, sem_r0, sem_i1, sem_r1, acc_sh):
    c = lax.axis_index("c")
    s = lax.axis_index("s")

    @pl.when(s < NS - 1)
    def _():
        pltpu.sync_copy(
            z_hbm.at[pl.ds(s * ROWS_MAIN, ROWS_MAIN)],
            acc_sh.at[pl.ds(s * ROWS_MAIN, ROWS_MAIN)],
        )

    @pl.when(s == NS - 1)
    def _():
        pltpu.sync_copy(
            z_hbm.at[pl.ds((NS - 1) * ROWS_MAIN, ROWS_TAIL)],
            acc_sh.at[pl.ds((NS - 1) * ROWS_MAIN, ROWS_TAIL)],
        )

    plsc.subcore_barrier()

    base0 = c * EDGES_PER_CORE + s * EDGES_PER_SUB

    pltpu.sync_copy(i_hbm.at[pl.ds(base0, TAIL)], idx_t)
    pltpu.sync_copy(t_hbm.at[pl.ds(base0, TAIL)], rows0.at[pl.ds(0, TAIL)])
    pltpu.sync_copy(rows0.at[pl.ds(0, TAIL)], acc_sh.at[idx_t], add=True)

    bufs = ((idx0, rows0, sem_i0, sem_r0), (idx1, rows1, sem_i1, sem_r1))

    def start_load(k, idx_v, rows_v, sem_i, sem_r):
        base = base0 + TAIL + k * CHUNK
        pltpu.async_copy(i_hbm.at[pl.ds(base, CHUNK)], idx_v, sem_i)
        pltpu.async_copy(t_hbm.at[pl.ds(base, CHUNK)], rows_v, sem_r)

    def wait_load(k, idx_v, rows_v, sem_i, sem_r):
        base = base0 + TAIL + k * CHUNK
        pltpu.make_async_copy(i_hbm.at[pl.ds(base, CHUNK)], idx_v, sem_i).wait()
        pltpu.make_async_copy(t_hbm.at[pl.ds(base, CHUNK)], rows_v, sem_r).wait()

    start_load(0, *bufs[0])

    def pair(p, _):
        k0 = 2 * p
        for b in range(2):
            k = k0 + b
            idx_v, rows_v, sem_i, sem_r = bufs[b]
            wait_load(k, idx_v, rows_v, sem_i, sem_r)

            @pl.when(k + 1 < NUM_CHUNKS)
            def _():
                start_load(k + 1, *bufs[1 - b])

            pltpu.sync_copy(rows_v, acc_sh.at[idx_v], add=True)
        return _

    lax.fori_loop(0, NUM_CHUNKS // 2, pair, None)

    plsc.subcore_barrier()

    @pl.when(s < NS - 1)
    def _():
        pltpu.sync_copy(
            acc_sh.at[pl.ds(s * ROWS_MAIN, ROWS_MAIN)],
            out_hbm.at[c, pl.ds(s * ROWS_MAIN, ROWS_MAIN)],
        )

    @pl.when(s == NS - 1)
    def _():
        pltpu.sync_copy(
            acc_sh.at[pl.ds((NS - 1) * ROWS_MAIN, ROWS_TAIL)],
            out_hbm.at[c, pl.ds((NS - 1) * ROWS_MAIN, ROWS_TAIL)],
        )


_scatter_stage = pl.kernel(
    _scatter_body,
    out_type=jax.ShapeDtypeStruct((NC, NUM_NODES, HIDDEN), jnp.float32),
    mesh=plsc.VectorSubcoreMesh(core_axis_name="c", subcore_axis_name="s"),
    scratch_types=[
        pltpu.VMEM((CHUNK,), jnp.int32),
        pltpu.VMEM((CHUNK, HIDDEN), jnp.float32),
        pltpu.VMEM((CHUNK,), jnp.int32),
        pltpu.VMEM((CHUNK, HIDDEN), jnp.float32),
        pltpu.VMEM((TAIL,), jnp.int32),
        pltpu.SemaphoreType.DMA,
        pltpu.SemaphoreType.DMA,
        pltpu.SemaphoreType.DMA,
        pltpu.SemaphoreType.DMA,
        pltpu.VMEM_SHARED((NUM_NODES, HIDDEN), jnp.float32),
    ],
)


def _mlp_body(parts_ref, wup_ref, wl_ref, bl_ref, wout_ref, out_ref):
    xt = parts_ref[0] + parts_ref[1]
    h = lax.dot_general(
        xt, wup_ref[...], (((1,), (1,)), ((), ())),
        preferred_element_type=jnp.float32,
    )
    for l in range(wl_ref.shape[0]):
        z = lax.dot_general(
            h, wl_ref[l], (((1,), (1,)), ((), ())),
            preferred_element_type=jnp.float32,
        ) + bl_ref[l][None, :]
        h = z * jax.nn.sigmoid(z)
    out_ref[...] = lax.dot_general(
        h, wout_ref[...], (((1,), (1,)), ((), ())),
        preferred_element_type=jnp.float32,
    )


def _mlp_stage(parts, w_up, w_layers, b_layers, w_out):
    grid = (NUM_NODES // NODE_BLOCK,)
    return pl.pallas_call(
        _mlp_body,
        grid=grid,
        in_specs=[
            pl.BlockSpec((NC, NODE_BLOCK, HIDDEN), lambda j: (0, j, 0)),
            pl.BlockSpec(w_up.shape, lambda j: (0, 0)),
            pl.BlockSpec(w_layers.shape, lambda j: (0, 0, 0)),
            pl.BlockSpec(b_layers.shape, lambda j: (0, 0)),
            pl.BlockSpec(w_out.shape, lambda j: (0, 0)),
        ],
        out_specs=pl.BlockSpec((NODE_BLOCK, w_out.shape[0]), lambda j: (j, 0)),
        out_shape=jax.ShapeDtypeStruct((NUM_NODES, w_out.shape[0]), jnp.float32),
    )(parts, w_up, w_layers, b_layers, w_out)


def kernel(x, rbf, i, num_nodes, W_rbfs, W_up, W_layers, b_layers, W_out):
    wt = jnp.transpose(W_rbfs[-1])
    t = _edge_stage(jnp.transpose(rbf), x, wt)
    zeros = jnp.zeros((NUM_NODES, HIDDEN), jnp.float32)
    parts = _scatter_stage(t, i, zeros)
    return _mlp_stage(parts, W_up, W_layers, b_layers, W_out)
